# baseline (speedup 1.0000x reference)
import functools

import jax
import jax.numpy as jnp
from jax import lax
from jax.experimental import pallas as pl
from jax.experimental.pallas import tpu as pltpu
from jax.experimental.pallas import tpu_sc as plsc

N = 50000
E = 160000
R = 3
D = 128

NC = 2
NS = 16
NCHUNK = 4
CW = D // NCHUNK

Npad = 50176
ROWS_PER_TILE = Npad // NS
K = 128
NSTEP = 80
EPT = NSTEP * K
Epad = EPT * NS
EROWS = R * NS * NSTEP
NBUF = 4

DTILES = 10
DSTEPS = Epad // (DTILES * K)

BN = 512
GRID_N = Npad // BN

_mesh = plsc.VectorSubcoreMesh(
    core_axis_name="c", subcore_axis_name="s", num_cores=NC, num_subcores=NS)


@functools.partial(
    pl.kernel,
    out_type=jax.ShapeDtypeStruct((NC * NS, Npad), jnp.float32),
    mesh=_mesh,
    compiler_params=pltpu.CompilerParams(use_tc_tiling_on_sc=False, needs_layout_passes=False),
    scratch_types=[
        pltpu.VMEM((DSTEPS * K,), jnp.int32),
        pltpu.VMEM((Npad,), jnp.float32),
    ],
)
def _sc_degrees(dst_hbm, zerosN_hbm, part_hbm, dst_v, hist):
    sc = lax.axis_index("c")
    tile = lax.axis_index("s")
    w = sc * NS + tile

    @pl.when(w < R * DTILES)
    def _():
        r = w // DTILES
        wl = w - r * DTILES
        eoff = (r * NS * NSTEP + wl * DSTEPS) * K
        pltpu.sync_copy(dst_hbm.at[pl.ds(eoff, DSTEPS * K)], dst_v)
        pltpu.sync_copy(zerosN_hbm, hist)
        ones16 = jnp.ones((16,), jnp.float32)

        def step(i, _):
            for jj in range(K // 16):
                idx = dst_v[pl.ds(i * K + jj * 16, 16)]
                plsc.addupdate_scatter(hist, [idx], ones16)
            return 0

        lax.fori_loop(0, DSTEPS, step, 0)
        pltpu.sync_copy(hist, part_hbm.at[w])


@functools.partial(
    pl.kernel,
    out_type=jax.ShapeDtypeStruct((R, NCHUNK, Npad, CW), jnp.float32),
    mesh=_mesh,
    compiler_params=pltpu.CompilerParams(use_tc_tiling_on_sc=False, needs_layout_passes=False),
    scratch_types=[
        pltpu.VMEM((NSTEP // 4 * K,), jnp.int32),
        pltpu.VMEM((NSTEP // 4 * K,), jnp.int32),
        pltpu.VMEM((5 * K, CW), jnp.float32),
        pltpu.VMEM_SHARED((Npad, CW), jnp.float32),
        pltpu.SemaphoreType.DMA,
    ],
)
def _sc_aggregate(hp_hbm, src_hbm, dst_hbm, zeros32_hbm, agg_hbm,
                  src_v, dst_v, rows, acc_sh, gsem):
    sc = lax.axis_index("c")
    tile = lax.axis_index("s")
    row0 = tile * ROWS_PER_TILE

    for p in range(R * NCHUNK // NC):
        for scid in range(NC):
            g = p * NC + scid
            r, c = divmod(g, NCHUNK)

            @pl.when(sc == scid)
            def _():
                pltpu.sync_copy(zeros32_hbm,
                                acc_sh.at[pl.ds(row0, ROWS_PER_TILE), :])
                plsc.subcore_barrier()
                QSTEP = NSTEP // 4

                for q in range(4):
                    eoff = ((r * NS + tile) * NSTEP + q * QSTEP) * K
                    pltpu.sync_copy(src_hbm.at[pl.ds(eoff, QSTEP * K)], src_v)
                    pltpu.sync_copy(dst_hbm.at[pl.ds(eoff, QSTEP * K)], dst_v)

                    def blk(bi, _):
                        j = bi * 5 * K
                        pltpu.async_copy(
                            hp_hbm.at[c].at[src_v.at[pl.ds(j, 5 * K)]], rows,
                            gsem).wait()
                        pltpu.sync_copy(rows,
                                        acc_sh.at[dst_v.at[pl.ds(j, 5 * K)]],
                                        add=True)
                        return 0

                    lax.fori_loop(0, QSTEP // 5, blk, 0)

                plsc.subcore_barrier()
                pltpu.sync_copy(
                    acc_sh.at[pl.ds(row0, ROWS_PER_TILE), :],
                    agg_hbm.at[r, c, pl.ds(row0, ROWS_PER_TILE), :])
                plsc.subcore_barrier()


def _repack_body(x_ref, o_ref):
    x = x_ref[...]
    for c in range(NCHUNK):
        o_ref[c] = x[:, c * CW:(c + 1) * CW]


def _repack(x):
    return pl.pallas_call(
        _repack_body,
        grid=(GRID_N,),
        in_specs=[pl.BlockSpec((BN, D), lambda i: (i, 0))],
        out_specs=pl.BlockSpec((NCHUNK, BN, CW), lambda i: (0, i, 0)),
        out_shape=jax.ShapeDtypeStruct((NCHUNK, Npad, CW), jnp.float32),
    )(x)


def _normred_body(part_ref, o_ref):
    p = part_ref[...]
    for r in range(R):
        deg = jnp.sum(p[r * DTILES:(r + 1) * DTILES], axis=0)
        norm = 1.0 / jnp.maximum(deg, 1.0)
        o_ref[r] = jnp.broadcast_to(norm[:, None], (BN, 8))


def _normred(part):
    return pl.pallas_call(
        _normred_body,
        grid=(GRID_N,),
        in_specs=[pl.BlockSpec((NC * NS, BN), lambda i: (0, i))],
        out_specs=pl.BlockSpec((R, BN, 8), lambda i: (0, i, 0)),
        out_shape=jax.ShapeDtypeStruct((R, Npad, 8), jnp.float32),
    )(part)


def _mm_body(packed, agg_ref, norm_ref, w_ref, b_ref, o_ref):
    acc = jnp.zeros((BN, D), jnp.float32)
    for r in range(R):
        norm = norm_ref[r, :, 0:1]
        a = jnp.concatenate([agg_ref[r, c] for c in range(NCHUNK)], axis=1)
        a = a * norm
        acc = acc + jnp.dot(a, w_ref[r], preferred_element_type=jnp.float32)
    b = b_ref[...]
    acc = acc + (b[0] + b[1] + b[2])[None, :]
    acc = jnp.maximum(acc, 0.0)
    if packed:
        for c in range(NCHUNK):
            o_ref[c] = acc[:, c * CW:(c + 1) * CW]
    else:
        o_ref[...] = acc


def _unpack_body(i_ref, o_ref):
    o_ref[...] = jnp.concatenate([i_ref[c] for c in range(NCHUNK)], axis=1)


def _unpack(hp):
    return pl.pallas_call(
        _unpack_body,
        grid=(GRID_N,),
        in_specs=[pl.BlockSpec((NCHUNK, BN, CW), lambda i: (0, i, 0))],
        out_specs=pl.BlockSpec((BN, D), lambda i: (i, 0)),
        out_shape=jax.ShapeDtypeStruct((Npad, D), jnp.float32),
    )(hp)


def _mm_layer(agg, norm, w, b):
    out_spec = pl.BlockSpec((NCHUNK, BN, CW), lambda i: (0, i, 0))
    out_shape = jax.ShapeDtypeStruct((NCHUNK, Npad, CW), jnp.float32)
    return pl.pallas_call(
        functools.partial(_mm_body, True),
        grid=(GRID_N,),
        in_specs=[
            pl.BlockSpec((R, NCHUNK, BN, CW), lambda i: (0, 0, i, 0)),
            pl.BlockSpec((R, BN, 8), lambda i: (0, i, 0)),
            pl.BlockSpec((R, D, D), lambda i: (0, 0, 0)),
            pl.BlockSpec((R, D), lambda i: (0, 0)),
        ],
        out_specs=out_spec,
        out_shape=out_shape,
    )(agg, norm, w, b)


def kernel(x, edge_index, W1, b1, W2, b2):
    ei = edge_index.astype(jnp.int32)
    src = jnp.pad(ei[:, 0, :], ((0, 0), (0, Epad - E))).reshape(-1)
    dst = jnp.pad(ei[:, 1, :], ((0, 0), (0, Epad - E)),
                  constant_values=N).reshape(-1)
    zerosN = jnp.zeros((Npad,), jnp.float32)
    zeros32 = jnp.zeros((ROWS_PER_TILE, CW), jnp.float32)

    part = _sc_degrees(dst, zerosN)
    norm = _normred(part)

    xpad = jnp.pad(x, ((0, Npad - N), (0, 0)))
    hp = _repack(xpad)

    def layer(h, wb):
        w, b = wb
        agg = _sc_aggregate(h, src, dst, zeros32)
        h2 = _mm_layer(agg, norm, w, b)
        return h2, 0.0

    hp2, _ = lax.scan(layer, hp, (jnp.stack([W1, W2]), jnp.stack([b1, b2])))
    out = _unpack(hp2)
    return out[:N]


# speedup vs baseline: 1.0471x; 1.0471x over previous
import functools

import jax
import jax.numpy as jnp
from jax import lax
from jax.experimental import pallas as pl
from jax.experimental.pallas import tpu as pltpu
from jax.experimental.pallas import tpu_sc as plsc

N = 50000
E = 160000
R = 3
D = 128

NC = 2
NS = 16
NCHUNK = 4
CW = D // NCHUNK

Npad = 50176
ROWS_PER_TILE = Npad // NS
K = 128
NSTEP = 80
EPT = NSTEP * K
Epad = EPT * NS
EROWS = R * NS * NSTEP
NBUF = 4

DTILES = 10
DSTEPS = Epad // (DTILES * K)

BN = 512
GRID_N = Npad // BN

_mesh = plsc.VectorSubcoreMesh(
    core_axis_name="c", subcore_axis_name="s", num_cores=NC, num_subcores=NS)


@functools.partial(
    pl.kernel,
    out_type=jax.ShapeDtypeStruct((NC * NS, Npad), jnp.float32),
    mesh=_mesh,
    compiler_params=pltpu.CompilerParams(use_tc_tiling_on_sc=False, needs_layout_passes=False),
    scratch_types=[
        pltpu.VMEM((DSTEPS * K,), jnp.int32),
        pltpu.VMEM((Npad,), jnp.float32),
    ],
)
def _sc_degrees(dst_hbm, zerosN_hbm, part_hbm, dst_v, hist):
    sc = lax.axis_index("c")
    tile = lax.axis_index("s")
    w = sc * NS + tile

    @pl.when(w < R * DTILES)
    def _():
        r = w // DTILES
        wl = w - r * DTILES
        eoff = (r * NS * NSTEP + wl * DSTEPS) * K
        pltpu.sync_copy(dst_hbm.at[pl.ds(eoff, DSTEPS * K)], dst_v)
        pltpu.sync_copy(zerosN_hbm, hist)
        ones16 = jnp.ones((16,), jnp.float32)

        def step(i, _):
            for jj in range(K // 16):
                idx = dst_v[pl.ds(i * K + jj * 16, 16)]
                plsc.addupdate_scatter(hist, [idx], ones16)
            return 0

        lax.fori_loop(0, DSTEPS, step, 0)
        pltpu.sync_copy(hist, part_hbm.at[w])


@functools.partial(
    pl.kernel,
    out_type=jax.ShapeDtypeStruct((R, NCHUNK, Npad, CW), jnp.float32),
    mesh=_mesh,
    compiler_params=pltpu.CompilerParams(use_tc_tiling_on_sc=False, needs_layout_passes=False),
    scratch_types=[
        pltpu.VMEM((NSTEP // 4 * K,), jnp.int32),
        pltpu.VMEM((NSTEP // 4 * K,), jnp.int32),
        [pltpu.VMEM((2 * K, CW), jnp.float32)] * 2,
        pltpu.VMEM_SHARED((Npad, CW), jnp.float32),
        [pltpu.SemaphoreType.DMA] * 2,
    ],
)
def _sc_aggregate(hp_hbm, src_hbm, dst_hbm, zeros32_hbm, agg_hbm,
                  src_v, dst_v, rows, acc_sh, gsems):
    sc = lax.axis_index("c")
    tile = lax.axis_index("s")
    row0 = tile * ROWS_PER_TILE

    for p in range(R * NCHUNK // NC):
        for scid in range(NC):
            g = p * NC + scid
            r, c = divmod(g, NCHUNK)

            @pl.when(sc == scid)
            def _():
                pltpu.sync_copy(zeros32_hbm,
                                acc_sh.at[pl.ds(row0, ROWS_PER_TILE), :])
                plsc.subcore_barrier()
                QSTEP = NSTEP // 4

                for q in range(4):
                    eoff = ((r * NS + tile) * NSTEP + q * QSTEP) * K
                    pltpu.sync_copy(src_hbm.at[pl.ds(eoff, QSTEP * K)], src_v)
                    pltpu.sync_copy(dst_hbm.at[pl.ds(eoff, QSTEP * K)], dst_v)

                    BLK = 2 * K
                    NBLK = QSTEP * K // BLK

                    def gath(j, b):
                        pltpu.async_copy(
                            hp_hbm.at[c].at[src_v.at[pl.ds(j * BLK, BLK)]],
                            rows[b], gsems[b])

                    def gath_wait(j, b):
                        pltpu.make_async_copy(
                            hp_hbm.at[c].at[src_v.at[pl.ds(j * BLK, BLK)]],
                            rows[b], gsems[b]).wait()

                    def scat(j, b):
                        pltpu.sync_copy(
                            rows[b], acc_sh.at[dst_v.at[pl.ds(j * BLK, BLK)]],
                            add=True)

                    gath(0, 0)
                    gath(1, 1)

                    def group(gi, _):
                        j = gi * 2
                        for b in range(2):
                            gath_wait(j + b, b)
                            scat(j + b, b)
                            gath(j + 2 + b, b)
                        return 0

                    lax.fori_loop(0, NBLK // 2 - 1, group, 0)
                    for b in range(2):
                        j = NBLK - 2 + b
                        gath_wait(j, b)
                        scat(j, b)

                plsc.subcore_barrier()
                pltpu.sync_copy(
                    acc_sh.at[pl.ds(row0, ROWS_PER_TILE), :],
                    agg_hbm.at[r, c, pl.ds(row0, ROWS_PER_TILE), :])
                plsc.subcore_barrier()


def _repack_body(x_ref, o_ref):
    x = x_ref[...]
    for c in range(NCHUNK):
        o_ref[c] = x[:, c * CW:(c + 1) * CW]


def _repack(x):
    return pl.pallas_call(
        _repack_body,
        grid=(GRID_N,),
        in_specs=[pl.BlockSpec((BN, D), lambda i: (i, 0))],
        out_specs=pl.BlockSpec((NCHUNK, BN, CW), lambda i: (0, i, 0)),
        out_shape=jax.ShapeDtypeStruct((NCHUNK, Npad, CW), jnp.float32),
    )(x)


def _normred_body(part_ref, o_ref):
    p = part_ref[...]
    for r in range(R):
        deg = jnp.sum(p[r * DTILES:(r + 1) * DTILES], axis=0)
        norm = 1.0 / jnp.maximum(deg, 1.0)
        o_ref[r] = jnp.broadcast_to(norm[:, None], (BN, 8))


def _normred(part):
    return pl.pallas_call(
        _normred_body,
        grid=(GRID_N,),
        in_specs=[pl.BlockSpec((NC * NS, BN), lambda i: (0, i))],
        out_specs=pl.BlockSpec((R, BN, 8), lambda i: (0, i, 0)),
        out_shape=jax.ShapeDtypeStruct((R, Npad, 8), jnp.float32),
    )(part)


def _mm_body(packed, agg_ref, norm_ref, w_ref, b_ref, o_ref):
    acc = jnp.zeros((BN, D), jnp.float32)
    for r in range(R):
        norm = norm_ref[r, :, 0:1]
        a = jnp.concatenate([agg_ref[r, c] for c in range(NCHUNK)], axis=1)
        a = a * norm
        acc = acc + jnp.dot(a, w_ref[r], preferred_element_type=jnp.float32)
    b = b_ref[...]
    acc = acc + (b[0] + b[1] + b[2])[None, :]
    acc = jnp.maximum(acc, 0.0)
    if packed:
        for c in range(NCHUNK):
            o_ref[c] = acc[:, c * CW:(c + 1) * CW]
    else:
        o_ref[...] = acc


def _unpack_body(i_ref, o_ref):
    o_ref[...] = jnp.concatenate([i_ref[c] for c in range(NCHUNK)], axis=1)


def _unpack(hp):
    return pl.pallas_call(
        _unpack_body,
        grid=(GRID_N,),
        in_specs=[pl.BlockSpec((NCHUNK, BN, CW), lambda i: (0, i, 0))],
        out_specs=pl.BlockSpec((BN, D), lambda i: (i, 0)),
        out_shape=jax.ShapeDtypeStruct((Npad, D), jnp.float32),
    )(hp)


def _mm_layer(agg, norm, w, b):
    out_spec = pl.BlockSpec((NCHUNK, BN, CW), lambda i: (0, i, 0))
    out_shape = jax.ShapeDtypeStruct((NCHUNK, Npad, CW), jnp.float32)
    return pl.pallas_call(
        functools.partial(_mm_body, True),
        grid=(GRID_N,),
        in_specs=[
            pl.BlockSpec((R, NCHUNK, BN, CW), lambda i: (0, 0, i, 0)),
            pl.BlockSpec((R, BN, 8), lambda i: (0, i, 0)),
            pl.BlockSpec((R, D, D), lambda i: (0, 0, 0)),
            pl.BlockSpec((R, D), lambda i: (0, 0)),
        ],
        out_specs=out_spec,
        out_shape=out_shape,
    )(agg, norm, w, b)


def kernel(x, edge_index, W1, b1, W2, b2):
    ei = edge_index.astype(jnp.int32)
    src = jnp.pad(ei[:, 0, :], ((0, 0), (0, Epad - E))).reshape(-1)
    dst = jnp.pad(ei[:, 1, :], ((0, 0), (0, Epad - E)),
                  constant_values=N).reshape(-1)
    zerosN = jnp.zeros((Npad,), jnp.float32)
    zeros32 = jnp.zeros((ROWS_PER_TILE, CW), jnp.float32)

    part = _sc_degrees(dst, zerosN)
    norm = _normred(part)

    xpad = jnp.pad(x, ((0, Npad - N), (0, 0)))
    hp = _repack(xpad)

    def layer(h, wb):
        w, b = wb
        agg = _sc_aggregate(h, src, dst, zeros32)
        h2 = _mm_layer(agg, norm, w, b)
        return h2, 0.0

    hp2, _ = lax.scan(layer, hp, (jnp.stack([W1, W2]), jnp.stack([b1, b2])))
    out = _unpack(hp2)
    return out[:N]


# speedup vs baseline: 1.0904x; 1.0414x over previous
import functools

import jax
import jax.numpy as jnp
from jax import lax
from jax.experimental import pallas as pl
from jax.experimental.pallas import tpu as pltpu
from jax.experimental.pallas import tpu_sc as plsc

N = 50000
E = 160000
R = 3
D = 128

NC = 2
NS = 16
NCHUNK = 4
CW = D // NCHUNK

Npad = 50176
ROWS_PER_TILE = Npad // NS
K = 128
NSTEP = 80
EPT = NSTEP * K
Epad = EPT * NS
EROWS = R * NS * NSTEP
NBUF = 4

DTILES = 10
DSTEPS = Epad // (DTILES * K)

BN = 512
GRID_N = Npad // BN

_mesh = plsc.VectorSubcoreMesh(
    core_axis_name="c", subcore_axis_name="s", num_cores=NC, num_subcores=NS)


@functools.partial(
    pl.kernel,
    out_type=jax.ShapeDtypeStruct((NC * NS, Npad), jnp.float32),
    mesh=_mesh,
    compiler_params=pltpu.CompilerParams(use_tc_tiling_on_sc=False, needs_layout_passes=False),
    scratch_types=[
        pltpu.VMEM((DSTEPS, K), jnp.int32),
        pltpu.VMEM((Npad,), jnp.float32),
    ],
)
def _sc_degrees(dst_hbm, zerosN_hbm, part_hbm, dst_v, hist):
    sc = lax.axis_index("c")
    tile = lax.axis_index("s")
    w = sc * NS + tile

    @pl.when(w < R * DTILES)
    def _():
        r = w // DTILES
        wl = w - r * DTILES
        row_e = r * NS * NSTEP + wl * DSTEPS
        pltpu.sync_copy(dst_hbm.at[pl.ds(row_e, DSTEPS), :], dst_v)
        pltpu.sync_copy(zerosN_hbm, hist)
        ones16 = jnp.ones((16,), jnp.float32)

        def step(i, _):
            for jj in range(K // 16):
                idx = dst_v[i, pl.ds(jj * 16, 16)]
                plsc.addupdate_scatter(hist, [idx], ones16)
            return 0

        lax.fori_loop(0, DSTEPS, step, 0)
        pltpu.sync_copy(hist, part_hbm.at[w])


@functools.partial(
    pl.kernel,
    out_type=jax.ShapeDtypeStruct((R, NCHUNK, Npad, CW), jnp.float32),
    mesh=_mesh,
    compiler_params=pltpu.CompilerParams(use_tc_tiling_on_sc=False, needs_layout_passes=False),
    scratch_types=[
        pltpu.VMEM((NSTEP // 4, K), jnp.int32),
        pltpu.VMEM((NSTEP // 4, K), jnp.int32),
        [pltpu.VMEM((K, CW), jnp.float32)] * NBUF,
        pltpu.VMEM_SHARED((Npad, CW), jnp.float32),
        [pltpu.SemaphoreType.DMA] * NBUF,
    ],
)
def _sc_aggregate(hp_hbm, src_hbm, dst_hbm, zeros32_hbm, agg_hbm,
                  src_v, dst_v, rows, acc_sh, gsems):
    sc = lax.axis_index("c")
    tile = lax.axis_index("s")
    row0 = tile * ROWS_PER_TILE

    for p in range(R * NCHUNK // NC):
        for scid in range(NC):
            g = p * NC + scid
            r, c = divmod(g, NCHUNK)

            @pl.when(sc == scid)
            def _():
                pltpu.sync_copy(zeros32_hbm,
                                acc_sh.at[pl.ds(row0, ROWS_PER_TILE), :])
                plsc.subcore_barrier()
                QSTEP = NSTEP // 4

                for q in range(4):
                    row_e = (r * NS + tile) * NSTEP + q * QSTEP
                    pltpu.sync_copy(src_hbm.at[pl.ds(row_e, QSTEP), :], src_v)
                    pltpu.sync_copy(dst_hbm.at[pl.ds(row_e, QSTEP), :], dst_v)

                    for b in range(NBUF):
                        pltpu.async_copy(hp_hbm.at[c].at[src_v.at[b]], rows[b],
                                         gsems[b])

                    def group(gi, _):
                        j = gi * NBUF
                        for b in range(NBUF):
                            pltpu.make_async_copy(
                                hp_hbm.at[c].at[src_v.at[j + b]], rows[b],
                                gsems[b]).wait()
                            pltpu.sync_copy(rows[b],
                                            acc_sh.at[dst_v.at[j + b]],
                                            add=True)
                            pltpu.async_copy(
                                hp_hbm.at[c].at[src_v.at[j + NBUF + b]],
                                rows[b], gsems[b])
                        return 0

                    lax.fori_loop(0, QSTEP // NBUF - 1, group, 0)
                    for b in range(NBUF):
                        j = QSTEP - NBUF + b
                        pltpu.make_async_copy(
                            hp_hbm.at[c].at[src_v.at[j]], rows[b],
                            gsems[b]).wait()
                        pltpu.sync_copy(rows[b], acc_sh.at[dst_v.at[j]],
                                        add=True)

                plsc.subcore_barrier()
                pltpu.sync_copy(
                    acc_sh.at[pl.ds(row0, ROWS_PER_TILE), :],
                    agg_hbm.at[r, c, pl.ds(row0, ROWS_PER_TILE), :])
                plsc.subcore_barrier()


def _repack_body(x_ref, o_ref):
    x = x_ref[...]
    for c in range(NCHUNK):
        o_ref[c] = x[:, c * CW:(c + 1) * CW]


def _repack(x):
    return pl.pallas_call(
        _repack_body,
        grid=(GRID_N,),
        in_specs=[pl.BlockSpec((BN, D), lambda i: (i, 0))],
        out_specs=pl.BlockSpec((NCHUNK, BN, CW), lambda i: (0, i, 0)),
        out_shape=jax.ShapeDtypeStruct((NCHUNK, Npad, CW), jnp.float32),
    )(x)


def _normred_body(part_ref, o_ref):
    p = part_ref[...]
    for r in range(R):
        deg = jnp.sum(p[r * DTILES:(r + 1) * DTILES], axis=0)
        norm = 1.0 / jnp.maximum(deg, 1.0)
        o_ref[r] = jnp.broadcast_to(norm[:, None], (BN, 8))


def _normred(part):
    return pl.pallas_call(
        _normred_body,
        grid=(GRID_N,),
        in_specs=[pl.BlockSpec((NC * NS, BN), lambda i: (0, i))],
        out_specs=pl.BlockSpec((R, BN, 8), lambda i: (0, i, 0)),
        out_shape=jax.ShapeDtypeStruct((R, Npad, 8), jnp.float32),
    )(part)


def _mm_body(packed, agg_ref, norm_ref, w_ref, b_ref, o_ref):
    acc = jnp.zeros((BN, D), jnp.float32)
    for r in range(R):
        norm = norm_ref[r, :, 0:1]
        a = jnp.concatenate([agg_ref[r, c] for c in range(NCHUNK)], axis=1)
        a = a * norm
        acc = acc + jnp.dot(a, w_ref[r], preferred_element_type=jnp.float32)
    b = b_ref[...]
    acc = acc + (b[0] + b[1] + b[2])[None, :]
    acc = jnp.maximum(acc, 0.0)
    if packed:
        for c in range(NCHUNK):
            o_ref[c] = acc[:, c * CW:(c + 1) * CW]
    else:
        o_ref[...] = acc


def _unpack_body(i_ref, o_ref):
    o_ref[...] = jnp.concatenate([i_ref[c] for c in range(NCHUNK)], axis=1)


def _unpack(hp):
    return pl.pallas_call(
        _unpack_body,
        grid=(GRID_N,),
        in_specs=[pl.BlockSpec((NCHUNK, BN, CW), lambda i: (0, i, 0))],
        out_specs=pl.BlockSpec((BN, D), lambda i: (i, 0)),
        out_shape=jax.ShapeDtypeStruct((Npad, D), jnp.float32),
    )(hp)


def _mm_layer(agg, norm, w, b):
    out_spec = pl.BlockSpec((NCHUNK, BN, CW), lambda i: (0, i, 0))
    out_shape = jax.ShapeDtypeStruct((NCHUNK, Npad, CW), jnp.float32)
    return pl.pallas_call(
        functools.partial(_mm_body, True),
        grid=(GRID_N,),
        in_specs=[
            pl.BlockSpec((R, NCHUNK, BN, CW), lambda i: (0, 0, i, 0)),
            pl.BlockSpec((R, BN, 8), lambda i: (0, i, 0)),
            pl.BlockSpec((R, D, D), lambda i: (0, 0, 0)),
            pl.BlockSpec((R, D), lambda i: (0, 0)),
        ],
        out_specs=out_spec,
        out_shape=out_shape,
    )(agg, norm, w, b)


def kernel(x, edge_index, W1, b1, W2, b2):
    ei = edge_index.astype(jnp.int32)
    src = jnp.pad(ei[:, 0, :], ((0, 0), (0, Epad - E))).reshape(EROWS, K)
    dst = jnp.pad(ei[:, 1, :], ((0, 0), (0, Epad - E)),
                  constant_values=N).reshape(EROWS, K)
    zerosN = jnp.zeros((Npad,), jnp.float32)
    zeros32 = jnp.zeros((ROWS_PER_TILE, CW), jnp.float32)

    part = _sc_degrees(dst, zerosN)
    norm = _normred(part)

    xpad = jnp.pad(x, ((0, Npad - N), (0, 0)))
    hp = _repack(xpad)

    def layer(h, wb):
        w, b = wb
        agg = _sc_aggregate(h, src, dst, zeros32)
        h2 = _mm_layer(agg, norm, w, b)
        return h2, 0.0

    hp2, _ = lax.scan(layer, hp, (jnp.stack([W1, W2]), jnp.stack([b1, b2])))
    out = _unpack(hp2)
    return out[:N]
